# baseline (speedup 1.0000x reference)
import functools

import jax
import jax.numpy as jnp
from jax import lax
from jax.experimental import pallas as pl
from jax.experimental.pallas import tpu as pltpu
from jax.experimental.pallas import tpu_sc as plsc

N = 100000
NP = 102400
D = 128
L = 6
NF = 12500
FACC = 12544
GW = 128
KB = 3
SPLIT = 81920
BR = 2048
CH = 128
EPS = 1e-5


def _sc_mesh():
    return plsc.VectorSubcoreMesh(core_axis_name="c", subcore_axis_name="s",
                                  num_cores=2, num_subcores=16)


def _sc_gather(h, idx_rows):
    n_out = idx_rows.shape[0] * KB * GW

    @functools.partial(
        pl.kernel,
        out_type=jax.ShapeDtypeStruct((n_out, D), jnp.float32),
        mesh=_sc_mesh(),
        scratch_types=[pltpu.SemaphoreType.DMA])
    def k(h_hbm, i_hbm, o_hbm, sem):
        def body(i_vmem, o_vmem):
            descs = [pltpu.async_copy(h_hbm.at[i_vmem.at[0, kk]],
                                      o_vmem.at[pl.ds(kk * GW, GW)], sem)
                     for kk in range(KB)]
            for dd in descs:
                dd.wait()

        pltpu.emit_pipeline(
            body,
            grid=(n_out // (KB * GW),),
            in_specs=[pl.BlockSpec((1, KB, GW), index_map=lambda i: (i, 0, 0))],
            out_specs=[pl.BlockSpec((KB * GW, D), index_map=lambda i: (i, 0))],
            core_axis_name=("c", "s"),
            dimension_semantics=(pltpu.PARALLEL,),
        )(i_hbm, o_hbm)

    return k(h, idx_rows)


def _tc_inproj(f, w, b, lg, lb):
    def body(f_ref, w_ref, b_ref, lg_ref, lb_ref, o_ref):
        x = jnp.dot(f_ref[...], w_ref[...],
                    preferred_element_type=jnp.float32) + b_ref[...]
        m = jnp.mean(x, axis=-1, keepdims=True)
        v = jnp.mean((x - m) ** 2, axis=-1, keepdims=True)
        y = (x - m) / jnp.sqrt(v + EPS) * lg_ref[...] + lb_ref[...]
        o_ref[...] = jnp.maximum(y, 0.0)

    fdim = f.shape[1]
    return pl.pallas_call(
        body,
        grid=(NP // BR,),
        in_specs=[pl.BlockSpec((BR, fdim), lambda i: (i, 0)),
                  pl.BlockSpec((fdim, D), lambda i: (0, 0)),
                  pl.BlockSpec((1, D), lambda i: (0, 0)),
                  pl.BlockSpec((1, D), lambda i: (0, 0)),
                  pl.BlockSpec((1, D), lambda i: (0, 0))],
        out_specs=pl.BlockSpec((BR, D), lambda i: (i, 0)),
        out_shape=jax.ShapeDtypeStruct((NP, D), jnp.float32),
    )(f, w, b, lg, lb)


def _tc_layer(h, g, w, b, lg, lb, half, dst):
    nb = (SPLIT if half == 0 else NP - SPLIT) // BR
    off = 0 if half == 0 else SPLIT // BR

    def body(h_ref, g1_ref, g2_ref, g3_ref, w_ref, b_ref, lg_ref, lb_ref,
             dst_ref, o_ref):
        x0 = h_ref[...]
        bf = jnp.bfloat16
        w16 = w_ref[...].astype(bf)
        acc = jnp.dot(x0.astype(bf), w16[0], preferred_element_type=jnp.float32)
        acc = acc + jnp.dot(g1_ref[...].astype(bf), w16[1],
                            preferred_element_type=jnp.float32)
        acc = acc + jnp.dot(g2_ref[...].astype(bf), w16[2],
                            preferred_element_type=jnp.float32)
        acc = acc + jnp.dot(g3_ref[...].astype(bf), w16[3],
                            preferred_element_type=jnp.float32)
        acc = acc + jnp.sum(b_ref[...], axis=0, keepdims=True)
        x = jnp.maximum(acc, 0.0)
        m = jnp.mean(x, axis=-1, keepdims=True)
        v = jnp.mean((x - m) ** 2, axis=-1, keepdims=True)
        o_ref[...] = (x - m) / jnp.sqrt(v + EPS) * lg_ref[...] + lb_ref[...] + x0

    in_specs = [pl.BlockSpec((BR, D), lambda i: (i + off, 0)),
                pl.BlockSpec((BR, D), lambda i: (i, 0)),
                pl.BlockSpec((BR, D), lambda i: (i + nb, 0)),
                pl.BlockSpec((BR, D), lambda i: (i + 2 * nb, 0)),
                pl.BlockSpec((4, D, D), lambda i: (0, 0, 0)),
                pl.BlockSpec((4, D), lambda i: (0, 0)),
                pl.BlockSpec((1, D), lambda i: (0, 0)),
                pl.BlockSpec((1, D), lambda i: (0, 0)),
                pl.BlockSpec(memory_space=pl.ANY)]
    args = [h, g, g, g, w, b, lg, lb, dst]
    return pl.pallas_call(
        body,
        grid=(nb,),
        in_specs=in_specs,
        out_specs=pl.BlockSpec((BR, D), lambda i: (i + off, 0)),
        out_shape=jax.ShapeDtypeStruct((NP, D), jnp.float32),
        input_output_aliases={8: 0},
    )(*args)


def _sc_segsum(emb, idx_rows, z128):
    RPS = FACC // 16
    rows_per_sub = NP // 32
    blks_per_sub = rows_per_sub // CH

    @functools.partial(
        pl.kernel,
        out_type=jax.ShapeDtypeStruct((2 * FACC, D), jnp.float32),
        mesh=_sc_mesh(),
        scratch_types=[pltpu.VMEM((CH, D), jnp.float32),
                       pltpu.VMEM((1, CH), jnp.int32),
                       pltpu.VMEM_SHARED((FACC, D), jnp.float32)])
    def k(emb_hbm, idx_hbm, z128_hbm, osum_hbm, emb_v, idx_v, acc_s):
        c = lax.axis_index("c")
        s = lax.axis_index("s")
        pltpu.sync_copy(z128_hbm.at[pl.ds(s * RPS, RPS)],
                        acc_s.at[pl.ds(s * RPS, RPS)])
        plsc.subcore_barrier()

        base_row = c * (NP // 2) + s * rows_per_sub
        base_blk = c * (NP // 2 // CH) + s * blks_per_sub

        @pl.loop(0, blks_per_sub)
        def _(i):
            pltpu.sync_copy(emb_hbm.at[pl.ds(base_row + i * CH, CH)], emb_v)
            pltpu.sync_copy(idx_hbm.at[pl.ds(base_blk + i, 1)], idx_v)
            pltpu.sync_copy(emb_v, acc_s.at[idx_v.at[0]], add=True)

        plsc.subcore_barrier()
        pltpu.sync_copy(acc_s.at[pl.ds(s * RPS, RPS)],
                        osum_hbm.at[pl.ds(c * FACC + s * RPS, RPS)])

    return k(emb, idx_rows, z128)


def _sc_segcount(idx_rows, ones16, z16):
    RPS = FACC // 16
    rows_per_sub = NP // 32
    blks_per_sub = rows_per_sub // CH

    @functools.partial(
        pl.kernel,
        out_type=jax.ShapeDtypeStruct((2 * FACC, D), jnp.float32),
        mesh=_sc_mesh(),
        scratch_types=[pltpu.VMEM((1, CH), jnp.int32),
                       pltpu.VMEM((CH, D), jnp.float32),
                       pltpu.VMEM_SHARED((FACC, D), jnp.float32)])
    def k(idx_hbm, ones_hbm, z16_hbm, ocnt_hbm, idx_v, ones_v, cnt_s):
        c = lax.axis_index("c")
        s = lax.axis_index("s")
        pltpu.sync_copy(z16_hbm.at[pl.ds(s * RPS, RPS)],
                        cnt_s.at[pl.ds(s * RPS, RPS)])
        pltpu.sync_copy(ones_hbm, ones_v)
        plsc.subcore_barrier()

        base_blk = c * (NP // 2 // CH) + s * blks_per_sub

        @pl.loop(0, blks_per_sub)
        def _(i):
            pltpu.sync_copy(idx_hbm.at[pl.ds(base_blk + i, 1)], idx_v)
            pltpu.sync_copy(ones_v, cnt_s.at[idx_v.at[0]], add=True)

        plsc.subcore_barrier()
        pltpu.sync_copy(cnt_s.at[pl.ds(s * RPS, RPS)],
                        ocnt_hbm.at[pl.ds(c * FACC + s * RPS, RPS)])

    return k(idx_rows, ones16, z16)


def _tc_pool(psum, pcnt, wo, bo, wa1, ba1, wa2row, ba2):
    def body(p_ref, c_ref, wo_ref, bo_ref, wa1_ref, ba1_ref, wa2_ref, ba2_ref,
             of_ref, og_ref):
        ssum = p_ref[:FACC] + p_ref[FACC:]
        cnt = c_ref[:FACC, 0:1] + c_ref[FACC:, 0:1]
        fe = jnp.dot(ssum / jnp.maximum(cnt, 1.0), wo_ref[...],
                     preferred_element_type=jnp.float32) + bo_ref[...]
        t = jnp.tanh(jnp.dot(fe, wa1_ref[...],
                             preferred_element_type=jnp.float32) + ba1_ref[...])
        gate = jnp.sum(t * wa2_ref[...], axis=-1, keepdims=True) + ba2_ref[0, 0]
        rid = lax.broadcasted_iota(jnp.int32, (FACC, 1), 0)
        valid = rid < NF
        gate = jnp.where(valid, gate, -jnp.inf)
        m = jnp.max(gate, axis=0, keepdims=True)
        e = jnp.where(valid, jnp.exp(gate - m), 0.0)
        attn = e / jnp.sum(e)
        of_ref[...] = fe[:NF]
        og_ref[...] = jnp.sum(attn * fe, axis=0, keepdims=True)

    def full(*shape):
        return pl.BlockSpec(shape, lambda: tuple(0 for _ in shape))

    return pl.pallas_call(
        body,
        in_specs=[full(2 * FACC, D), full(2 * FACC, D), full(D, D),
                  full(1, D), full(D, D), full(1, D), full(1, D),
                  full(1, 1)],
        out_specs=(pl.BlockSpec((NF, D), lambda: (0, 0)),
                   pl.BlockSpec((1, D), lambda: (0, 0))),
        out_shape=(jax.ShapeDtypeStruct((NF, D), jnp.float32),
                   jax.ShapeDtypeStruct((1, D), jnp.float32)),
    )(psum, pcnt, wo, bo, wa1, ba1, wa2row, ba2)


def kernel(features, next_indices, prev_indices, mate_indices, face_indices,
           W_in, b_in, ln_in_g, ln_in_b, Wc, bc, ln_g, ln_b,
           W_out, b_out, Wa1, ba1, Wa2, ba2):
    f32 = jnp.float32
    pad = NP - N
    f = jnp.pad(features.astype(f32), ((0, pad), (0, 0)))
    nxt = jnp.pad(next_indices.astype(jnp.int32), (0, pad))
    prv = jnp.pad(prev_indices.astype(jnp.int32), (0, pad))
    mte = jnp.pad(mate_indices.astype(jnp.int32), (0, pad))
    halves = []
    for lo, hi in ((0, SPLIT), (SPLIT, NP)):
        halves.append(jnp.concatenate([
            nxt[lo:hi], prv[lo:hi], mte[lo:hi],
        ]).reshape(3 * (hi - lo) // (KB * GW), KB, GW))
    idxA, idxB = halves

    fidx = jnp.pad(face_indices.astype(jnp.int32), (0, pad),
                   constant_values=NF).reshape(NP // CH, CH)
    pcnt = _sc_segcount(fidx, jnp.ones((CH, D), f32),
                        jnp.zeros((FACC, D), f32))
    h = _tc_inproj(f, W_in, b_in.reshape(1, D), ln_in_g.reshape(1, D),
                   ln_in_b.reshape(1, D))
    dsts = [jnp.zeros((NP, D), f32), jnp.zeros((NP, D), f32)]
    for l in range(L):
        gA = _sc_gather(h, idxA)
        gB = _sc_gather(h, idxB)
        dst = dsts.pop(0)
        part = _tc_layer(h, gA, Wc[l], bc[l], ln_g[l].reshape(1, D),
                         ln_b[l].reshape(1, D), 0, dst)
        newh = _tc_layer(h, gB, Wc[l], bc[l], ln_g[l].reshape(1, D),
                         ln_b[l].reshape(1, D), 1, part)
        dsts.append(h)
        h = newh

    psum = _sc_segsum(h, fidx, jnp.zeros((FACC, D), f32))
    face_emb, graph = _tc_pool(psum, pcnt, W_out, b_out.reshape(1, D),
                               Wa1, ba1.reshape(1, D),
                               Wa2.reshape(1, D), ba2.reshape(1, 1))
    return face_emb, graph.reshape(D)
